# Initial kernel scaffold; baseline (speedup 1.0000x reference)
import jax
import jax.numpy as jnp
from jax.experimental import pallas as pl


def kernel(x, edge_index, edge_attr, u, batch):
    raise NotImplementedError("write your pallas kernel here")


# speedup vs baseline: 9.2225x; 9.2225x over previous
import functools

import jax
import jax.numpy as jnp
from jax import lax
from jax.experimental import pallas as pl
from jax.experimental.pallas import tpu as pltpu


def _tc_body(x_ref, b_ref, ut_ref, o_ref):
    n, d = x_ref.shape
    g = ut_ref.shape[1]
    r = jnp.sum(x_ref[...], axis=1, keepdims=True)
    gid = lax.broadcasted_iota(jnp.int32, (n, g), 1)
    m = b_ref[...] == gid
    s = jnp.sum(jnp.where(m, r, 0.0), axis=0, keepdims=True)
    cnt = jnp.sum(m.astype(jnp.float32), axis=0, keepdims=True)
    us = jnp.sum(ut_ref[...], axis=0, keepdims=True)
    denom = jnp.float32(ut_ref.shape[0] + d)
    o_ref[...] = (us + s / jnp.maximum(cnt, 1.0)) / denom


def kernel(x, edge_index, edge_attr, u, batch):
    del edge_index, edge_attr
    n, d = x.shape
    g = u.shape[0]
    b2 = batch.astype(jnp.int32).reshape(n, 1)
    ut = u.T
    out = pl.pallas_call(
        _tc_body,
        out_shape=jax.ShapeDtypeStruct((1, g), jnp.float32),
    )(x, b2, ut)
    return out.reshape(g)
